# baseline (speedup 1.0000x reference)
import functools

import jax
import jax.numpy as jnp
from jax import lax
from jax.experimental import pallas as pl
from jax.experimental.pallas import tpu as pltpu
from jax.experimental.pallas import tpu_sc as plsc

WIDTH = 16
N_NODES = 10000
N_EDGES = 320000
D_EDGE = 16

NC = 2
NS = 16
NW = NC * NS
EW = N_EDGES // NW
ROW = 80
RPW = EW // ROW
CH_ROWS = 5
CH_E = CH_ROWS * ROW
NCHUNK = RPW // CH_ROWS
NPT = N_NODES // NS


def _sc_segsum_body(idx_hbm, attr_hbm, s_out, c_out,
                    idx_buf, vbuf, ones_buf, stage, s_sh, c_sh):
    cid = lax.axis_index("c")
    sid = lax.axis_index("s")
    wid = sid * NC + cid

    def fill(i, _):
        ones_buf[i, :] = jnp.ones((16,), jnp.float32)
        return 0
    lax.fori_loop(0, ROW, fill, 0)

    def zfill(i, _):
        stage[i, :] = jnp.zeros((16,), jnp.float32)
        return 0
    lax.fori_loop(0, NPT, zfill, 0)

    pltpu.sync_copy(stage, s_sh.at[pl.ds(sid * NPT, NPT)])
    pltpu.sync_copy(stage, c_sh.at[pl.ds(sid * NPT, NPT)])
    plsc.subcore_barrier()

    pltpu.sync_copy(idx_hbm.at[1, pl.ds(wid * EW, EW)], idx_buf)

    def chunk(k, _):
        e0 = wid * EW + k * CH_E
        pltpu.sync_copy(attr_hbm.at[pl.ds(e0, CH_E)], vbuf)
        for j in range(CH_ROWS):
            r = k * CH_E + j * ROW
            pltpu.sync_copy(vbuf.at[pl.ds(j * ROW, ROW)],
                            s_sh.at[idx_buf.at[pl.ds(r, ROW)]], add=True)
            pltpu.sync_copy(ones_buf, c_sh.at[idx_buf.at[pl.ds(r, ROW)]], add=True)
        return 0
    lax.fori_loop(0, NCHUNK, chunk, 0)

    plsc.subcore_barrier()

    pltpu.sync_copy(s_sh.at[pl.ds(sid * NPT, NPT)], stage)
    pltpu.sync_copy(stage, s_out.at[cid, sid])
    pltpu.sync_copy(c_sh.at[pl.ds(sid * NPT, NPT)], stage)
    pltpu.sync_copy(stage, c_out.at[cid, sid])


_sc_segsum = pl.kernel(
    _sc_segsum_body,
    out_type=(
        jax.ShapeDtypeStruct((NC, NS, NPT, D_EDGE), jnp.float32),
        jax.ShapeDtypeStruct((NC, NS, NPT, D_EDGE), jnp.float32),
    ),
    mesh=plsc.VectorSubcoreMesh(
        core_axis_name="c", subcore_axis_name="s",
        num_cores=NC, num_subcores=NS),
    compiler_params=pltpu.CompilerParams(use_tc_tiling_on_sc=False),
    scratch_types=[
        pltpu.VMEM((EW,), jnp.int32),
        pltpu.VMEM((CH_E, D_EDGE), jnp.float32),
        pltpu.VMEM((ROW, D_EDGE), jnp.float32),
        pltpu.VMEM((NPT, D_EDGE), jnp.float32),
        pltpu.VMEM_SHARED((N_NODES, D_EDGE), jnp.float32),
        pltpu.VMEM_SHARED((N_NODES, D_EDGE), jnp.float32),
    ],
)


def _finish_body(x_ref, s_ref, c_ref, w_ref, b_ref, o_ref):
    x = x_ref[...]
    s4 = s_ref[...]
    c4 = c_ref[...]
    S = jnp.sum(s4, axis=0).reshape(N_NODES, D_EDGE)
    cntw = jnp.sum(c4, axis=0).reshape(N_NODES, D_EDGE)
    W = w_ref[...]
    b2 = b_ref[...]
    rows = lax.broadcasted_iota(jnp.int32, (WIDTH * WIDTH, WIDTH), 0)
    cols = lax.broadcasted_iota(jnp.int32, (WIDTH * WIDTH, WIDTH), 1)
    sel = jnp.where(rows // WIDTH == cols, 1.0, 0.0)
    wrow = jnp.dot(W, sel, preferred_element_type=jnp.float32)
    brow = lax.dot_general(sel, b2, (((0,), (1,)), ((), ())),
                           preferred_element_type=jnp.float32)
    A = jnp.dot(S, wrow, preferred_element_type=jnp.float32)
    u = jnp.dot(x, brow, preferred_element_type=jnp.float32)
    term1 = jnp.sum(x * A)
    term2 = jnp.sum(cntw * u) * (1.0 / WIDTH)
    o_ref[0, 0] = (term1 + term2) * (1.0 / (N_EDGES * WIDTH))


@functools.partial(pl.pallas_call,
                   out_shape=jax.ShapeDtypeStruct((1, 1), jnp.float32),
                   out_specs=pl.BlockSpec(memory_space=pltpu.SMEM))
def _finish(x_ref, s_ref, c_ref, w_ref, b_ref, o_ref):
    _finish_body(x_ref, s_ref, c_ref, w_ref, b_ref, o_ref)


def kernel(x, edge_index, edge_attr, W_nn, b_nn):
    s2, c2 = _sc_segsum(edge_index, edge_attr)
    out = _finish(x, s2, c2, W_nn, b_nn.reshape(1, WIDTH * WIDTH))
    return out[0, 0]


# speedup vs baseline: 1.5209x; 1.5209x over previous
import functools

import jax
import jax.numpy as jnp
from jax import lax
from jax.experimental import pallas as pl
from jax.experimental.pallas import tpu as pltpu
from jax.experimental.pallas import tpu_sc as plsc

WIDTH = 16
N_NODES = 10000
N_EDGES = 320000
D_EDGE = 16

NC = 2
NS = 16
NW = NC * NS
EW = N_EDGES // NW
ROW = 125
RPW = EW // ROW
CH_ROWS = 8
CH_E = CH_ROWS * ROW
NCHUNK = RPW // CH_ROWS
NPT = N_NODES // NS


def _sc_segsum_body(idx_hbm, attrT_hbm, s_out, c_out,
                    idx_buf, tbuf, vbuf, ones_buf, stage, s_sh, c_sh,
                    sem_idx, sem_t0, sem_t1, sem_sc):
    cid = lax.axis_index("c")
    sid = lax.axis_index("s")
    wid = sid * NC + cid

    idx_cp = pltpu.async_copy(idx_hbm.at[wid], idx_buf, sem_idx)
    t_sems = (sem_t0, sem_t1)
    e_base = wid * EW
    pltpu.async_copy(attrT_hbm.at[:, pl.ds(e_base, CH_E)],
                     tbuf.at[0], sem_t0)
    pltpu.async_copy(attrT_hbm.at[:, pl.ds(e_base + CH_E, CH_E)],
                     tbuf.at[1], sem_t1)

    def fill(i, _):
        ones_buf[i, :] = jnp.ones((16,), jnp.float32)
        return 0
    lax.fori_loop(0, ROW, fill, 0)

    def zfill(i, _):
        stage[i, :] = jnp.zeros((16,), jnp.float32)
        return 0
    lax.fori_loop(0, NPT, zfill, 0)

    pltpu.sync_copy(stage, s_sh.at[pl.ds(sid * NPT, NPT)])
    pltpu.sync_copy(stage, c_sh.at[pl.ds(sid * NPT, NPT)])
    idx_cp.wait()
    plsc.subcore_barrier()

    rows16 = jax.lax.iota(jnp.int32, 16)

    def do_chunk(k, par):
        @pl.when(k + 2 < NCHUNK)
        def _():
            pltpu.async_copy(
                attrT_hbm.at[:, pl.ds(e_base + (k + 2) * CH_E, CH_E)],
                tbuf.at[1 - par], t_sems[1 - par])

        pltpu.make_async_copy(attrT_hbm.at[:, pl.ds(e_base, CH_E)],
                              tbuf.at[par], t_sems[par]).wait()

        def repack(e, _):
            cols = jnp.zeros((16,), jnp.int32) + e
            v = plsc.load_gather(tbuf.at[par], [rows16, cols])
            vbuf[e, :] = v
            return 0
        lax.fori_loop(0, CH_E, repack, 0)

        for j in range(CH_ROWS):
            r = k * CH_ROWS + j
            pltpu.async_copy(vbuf.at[pl.ds(j * ROW, ROW)],
                             s_sh.at[idx_buf.at[r]], sem_sc, add=True)
            pltpu.async_copy(ones_buf, c_sh.at[idx_buf.at[r]], sem_sc,
                             add=True)
        for j in range(CH_ROWS):
            r = k * CH_ROWS + j
            pltpu.make_async_copy(vbuf.at[pl.ds(j * ROW, ROW)],
                                  s_sh.at[idx_buf.at[r]], sem_sc).wait()
            pltpu.make_async_copy(ones_buf, c_sh.at[idx_buf.at[r]],
                                  sem_sc).wait()

    def chunk_pair(t, _):
        do_chunk(2 * t, 0)
        do_chunk(2 * t + 1, 1)
        return 0
    lax.fori_loop(0, NCHUNK // 2, chunk_pair, 0)

    plsc.subcore_barrier()

    pltpu.sync_copy(s_sh.at[pl.ds(sid * NPT, NPT)], stage)
    pltpu.sync_copy(stage, s_out.at[cid, sid])
    pltpu.sync_copy(c_sh.at[pl.ds(sid * NPT, NPT)], stage)
    pltpu.sync_copy(stage, c_out.at[cid, sid])


_sc_segsum = pl.kernel(
    _sc_segsum_body,
    out_type=(
        jax.ShapeDtypeStruct((NC, NS, NPT, D_EDGE), jnp.float32),
        jax.ShapeDtypeStruct((NC, NS, NPT, D_EDGE), jnp.float32),
    ),
    mesh=plsc.VectorSubcoreMesh(
        core_axis_name="c", subcore_axis_name="s",
        num_cores=NC, num_subcores=NS),
    compiler_params=pltpu.CompilerParams(use_tc_tiling_on_sc=False,
                                         needs_layout_passes=False),
    scratch_types=[
        pltpu.VMEM((RPW, ROW), jnp.int32),
        pltpu.VMEM((2, D_EDGE, CH_E), jnp.float32),
        pltpu.VMEM((CH_E, D_EDGE), jnp.float32),
        pltpu.VMEM((ROW, D_EDGE), jnp.float32),
        pltpu.VMEM((NPT, D_EDGE), jnp.float32),
        pltpu.VMEM_SHARED((N_NODES, D_EDGE), jnp.float32),
        pltpu.VMEM_SHARED((N_NODES, D_EDGE), jnp.float32),
        pltpu.SemaphoreType.DMA,
        pltpu.SemaphoreType.DMA,
        pltpu.SemaphoreType.DMA,
        pltpu.SemaphoreType.DMA,
    ],
)


def _finish_body(xt_ref, s_ref, c_ref, w_ref, b_ref, o_ref):
    xt = xt_ref[...]
    S = (s_ref[0] + s_ref[1]).reshape(N_NODES, D_EDGE)
    cntw = (c_ref[0] + c_ref[1]).reshape(N_NODES, D_EDGE)
    W = w_ref[...]
    b2 = b_ref[...]
    rows = lax.broadcasted_iota(jnp.int32, (WIDTH, WIDTH * WIDTH), 0)
    cols = lax.broadcasted_iota(jnp.int32, (WIDTH, WIDTH * WIDTH), 1)
    selT = jnp.where(cols // WIDTH == rows, 1.0, 0.0)
    wrowT = lax.dot_general(selT, W, (((1,), (1,)), ((), ())),
                            preferred_element_type=jnp.float32)
    browr = lax.dot_general(b2, selT, (((1,), (1,)), ((), ())),
                            preferred_element_type=jnp.float32)
    M = lax.dot_general(xt, S, (((1,), (0,)), ((), ())),
                        preferred_element_type=jnp.float32)
    V = lax.dot_general(xt, cntw, (((1,), (0,)), ((), ())),
                        preferred_element_type=jnp.float32)
    term1 = jnp.sum(M * wrowT)
    term2 = jnp.sum(lax.dot_general(browr, V, (((1,), (0,)), ((), ())),
                                    preferred_element_type=jnp.float32))
    term2 = term2 * (1.0 / WIDTH)
    o_ref[0, 0] = (term1 + term2) * (1.0 / (N_EDGES * WIDTH))


@functools.partial(pl.pallas_call,
                   out_shape=jax.ShapeDtypeStruct((1, 1), jnp.float32),
                   out_specs=pl.BlockSpec(memory_space=pltpu.SMEM))
def _finish(xt_ref, s_ref, c_ref, w_ref, b_ref, o_ref):
    _finish_body(xt_ref, s_ref, c_ref, w_ref, b_ref, o_ref)


def kernel(x, edge_index, edge_attr, W_nn, b_nn):
    idx3d = edge_index[1].reshape(NW, RPW, ROW)
    s2, c2 = _sc_segsum(idx3d, edge_attr.T)
    out = _finish(x.T, s2, c2, W_nn, b_nn.reshape(1, WIDTH * WIDTH))
    return out[0, 0]
